# baseline (device time: 47059 ns/iter reference)
import jax
import jax.numpy as jnp
from jax import lax
from jax.experimental import pallas as pl
from jax.experimental.pallas import tpu as pltpu

N_DEV = 32
B = 256
D = 256
RB = B // N_DEV
CHUNK = 8
N_CHUNKS = N_DEV // CHUNK


def kernel(x, Win0, Wout0, Win1, Wout1, Win2, Wout2):
    def body(x_ref, win0_ref, wout0_ref, win1_ref, wout1_ref, win2_ref,
             wout2_ref, out_ref, acc_ref, rs_ref, ag_ref,
             rs_send_sems, rs_recv_sems, ag_send_sems, ag_recv_sems):
        my = lax.axis_index("i")

        def rs_rdma(layer, j):
            return pltpu.make_async_remote_copy(
                src_ref=acc_ref.at[pl.ds(j * RB, RB), :],
                dst_ref=rs_ref.at[layer, my],
                send_sem=rs_send_sems.at[layer, j],
                recv_sem=rs_recv_sems.at[layer, my],
                device_id=(j,),
                device_id_type=pl.DeviceIdType.MESH,
            )

        def rs_recv(layer, j):
            return pltpu.make_async_remote_copy(
                src_ref=acc_ref.at[pl.ds(j * RB, RB), :],
                dst_ref=rs_ref.at[layer, j],
                send_sem=rs_send_sems.at[layer, j],
                recv_sem=rs_recv_sems.at[layer, j],
                device_id=(j,),
                device_id_type=pl.DeviceIdType.MESH,
            )

        def ag_rdma(layer, j):
            return pltpu.make_async_remote_copy(
                src_ref=ag_ref.at[layer, my],
                dst_ref=ag_ref.at[layer, my],
                send_sem=ag_send_sems.at[layer, j],
                recv_sem=ag_recv_sems.at[layer, my],
                device_id=(j,),
                device_id_type=pl.DeviceIdType.MESH,
            )

        def ag_recv(layer, j):
            return pltpu.make_async_remote_copy(
                src_ref=ag_ref.at[layer, my],
                dst_ref=ag_ref.at[layer, j],
                send_sem=ag_send_sems.at[layer, j],
                recv_sem=ag_recv_sems.at[layer, j],
                device_id=(j,),
                device_id_type=pl.DeviceIdType.MESH,
            )

        barrier_sem = pltpu.get_barrier_semaphore()
        for j in range(N_DEV):
            @pl.when(j != my)
            def _(j=j):
                pl.semaphore_signal(
                    barrier_sem, inc=1,
                    device_id=(j,), device_id_type=pl.DeviceIdType.MESH,
                )
        pl.semaphore_wait(barrier_sem, N_DEV - 1)

        wins = [win0_ref, win1_ref, win2_ref]
        wouts = [wout0_ref, wout1_ref, wout2_ref]

        h = jnp.maximum(
            jnp.dot(x_ref[...], wins[0][...],
                    preferred_element_type=jnp.float32),
            0.0,
        )
        acc_ref[...] = jnp.dot(h, wouts[0][...],
                               preferred_element_type=jnp.float32)
        for j in range(N_DEV):
            @pl.when(j != my)
            def _(j=j):
                rs_rdma(0, j).start()
        rs_ref[0, my] = acc_ref[pl.ds(my * RB, RB), :]

        for layer in range(1, 3):
            for j in range(N_DEV):
                @pl.when(j != my)
                def _(j=j):
                    r = rs_recv(layer - 1, j)
                    r.wait_recv()
                    r.wait_send()
            block = jnp.sum(rs_ref[layer - 1], axis=0)
            ag_ref[layer - 1, my] = block
            for j in range(N_DEV):
                @pl.when(j != my)
                def _(j=j):
                    ag_rdma(layer - 1, j).start()

            for c in range(N_CHUNKS):
                lo = c * CHUNK
                for j in range(lo, lo + CHUNK):
                    @pl.when(j != my)
                    def _(j=j):
                        ag_recv(layer - 1, j).wait_recv()
                x_chunk = ag_ref[layer - 1, lo:lo + CHUNK].reshape(
                    CHUNK * RB, D)
                hc = jnp.maximum(
                    jnp.dot(x_chunk, wins[layer][...],
                            preferred_element_type=jnp.float32),
                    0.0,
                )
                acc_ref[pl.ds(lo * RB, CHUNK * RB), :] = jnp.dot(
                    hc, wouts[layer][...], preferred_element_type=jnp.float32)
                for j in range(lo, lo + CHUNK):
                    @pl.when(j != my)
                    def _(j=j):
                        rs_rdma(layer, j).start()
                    @pl.when(j == my)
                    def _(j=j):
                        rs_ref[layer, my] = acc_ref[pl.ds(my * RB, RB), :]

            for j in range(N_DEV):
                @pl.when(j != my)
                def _(j=j):
                    ag_recv(layer - 1, j).wait_send()

        for j in range(N_DEV):
            @pl.when(j != my)
            def _(j=j):
                r = rs_recv(2, j)
                r.wait_recv()
                r.wait_send()
        out_ref[...] = jnp.sum(rs_ref[2], axis=0)

    return pl.pallas_call(
        body,
        out_shape=jax.ShapeDtypeStruct((RB, D), jnp.float32),
        in_specs=[pl.BlockSpec(memory_space=pltpu.VMEM)] * 7,
        out_specs=pl.BlockSpec(memory_space=pltpu.VMEM),
        scratch_shapes=[
            pltpu.VMEM((B, D), jnp.float32),
            pltpu.VMEM((3, N_DEV, RB, D), jnp.float32),
            pltpu.VMEM((2, N_DEV, RB, D), jnp.float32),
            pltpu.SemaphoreType.DMA((3, N_DEV)),
            pltpu.SemaphoreType.DMA((3, N_DEV)),
            pltpu.SemaphoreType.DMA((2, N_DEV)),
            pltpu.SemaphoreType.DMA((2, N_DEV)),
        ],
        compiler_params=pltpu.CompilerParams(collective_id=0),
    )(x, Win0, Wout0, Win1, Wout1, Win2, Wout2)


# device time: 46591 ns/iter; 1.0100x vs baseline; 1.0100x over previous
import jax
import jax.numpy as jnp
from jax import lax
from jax.experimental import pallas as pl
from jax.experimental.pallas import tpu as pltpu

N_DEV = 32
B = 256
D = 256
RB = B // N_DEV
CHUNK = 8
N_CHUNKS = N_DEV // CHUNK


def kernel(x, Win0, Wout0, Win1, Wout1, Win2, Wout2):
    def body(x_ref, win0_ref, wout0_ref, win1_ref, wout1_ref, win2_ref,
             wout2_ref, out_ref, acc_ref, rs_ref, ag_ref,
             rs_send_sems, rs_recv_sems, ag_send_sems, ag_recv_sems):
        my = lax.axis_index("i")

        def rs_rdma(layer, j):
            return pltpu.make_async_remote_copy(
                src_ref=acc_ref.at[pl.ds(j * RB, RB), :],
                dst_ref=rs_ref.at[layer, my],
                send_sem=rs_send_sems.at[layer, j],
                recv_sem=rs_recv_sems.at[layer, my],
                device_id=(j,),
                device_id_type=pl.DeviceIdType.MESH,
            )

        def rs_recv(layer, j):
            return pltpu.make_async_remote_copy(
                src_ref=acc_ref.at[pl.ds(j * RB, RB), :],
                dst_ref=rs_ref.at[layer, j],
                send_sem=rs_send_sems.at[layer, j],
                recv_sem=rs_recv_sems.at[layer, j],
                device_id=(j,),
                device_id_type=pl.DeviceIdType.MESH,
            )

        def ag_rdma(layer, j):
            return pltpu.make_async_remote_copy(
                src_ref=ag_ref.at[layer, my],
                dst_ref=ag_ref.at[layer, my],
                send_sem=ag_send_sems.at[layer, j],
                recv_sem=ag_recv_sems.at[layer, my],
                device_id=(j,),
                device_id_type=pl.DeviceIdType.MESH,
            )

        def ag_recv(layer, j):
            return pltpu.make_async_remote_copy(
                src_ref=ag_ref.at[layer, my],
                dst_ref=ag_ref.at[layer, j],
                send_sem=ag_send_sems.at[layer, j],
                recv_sem=ag_recv_sems.at[layer, j],
                device_id=(j,),
                device_id_type=pl.DeviceIdType.MESH,
            )

        wins = [win0_ref, win1_ref, win2_ref]
        wouts = [wout0_ref, wout1_ref, wout2_ref]

        barrier_sem = pltpu.get_barrier_semaphore()
        for j in range(N_DEV):
            @pl.when(j != my)
            def _(j=j):
                pl.semaphore_signal(
                    barrier_sem, inc=1,
                    device_id=(j,), device_id_type=pl.DeviceIdType.MESH,
                )

        h = jnp.maximum(
            jnp.dot(x_ref[...], wins[0][...],
                    preferred_element_type=jnp.float32),
            0.0,
        )
        acc_ref[...] = jnp.dot(h, wouts[0][...],
                               preferred_element_type=jnp.float32)

        pl.semaphore_wait(barrier_sem, N_DEV - 1)

        for j in range(N_DEV):
            @pl.when(j != my)
            def _(j=j):
                rs_rdma(0, j).start()
        rs_ref[0, my] = acc_ref[pl.ds(my * RB, RB), :]

        for layer in range(1, 3):
            for j in range(N_DEV):
                @pl.when(j != my)
                def _(j=j):
                    r = rs_recv(layer - 1, j)
                    r.wait_recv()
                    r.wait_send()
            block = jnp.sum(rs_ref[layer - 1], axis=0)
            ag_ref[layer - 1, my] = block
            for j in range(N_DEV):
                @pl.when(j != my)
                def _(j=j):
                    ag_rdma(layer - 1, j).start()

            for c in range(N_CHUNKS):
                lo = c * CHUNK
                for j in range(lo, lo + CHUNK):
                    @pl.when(j != my)
                    def _(j=j):
                        ag_recv(layer - 1, j).wait_recv()
                x_chunk = ag_ref[layer - 1, lo:lo + CHUNK].reshape(
                    CHUNK * RB, D)
                hc = jnp.maximum(
                    jnp.dot(x_chunk, wins[layer][...],
                            preferred_element_type=jnp.float32),
                    0.0,
                )
                acc_ref[pl.ds(lo * RB, CHUNK * RB), :] = jnp.dot(
                    hc, wouts[layer][...], preferred_element_type=jnp.float32)
                for j in range(lo, lo + CHUNK):
                    @pl.when(j != my)
                    def _(j=j):
                        rs_rdma(layer, j).start()
                    @pl.when(j == my)
                    def _(j=j):
                        rs_ref[layer, my] = acc_ref[pl.ds(my * RB, RB), :]

            for j in range(N_DEV):
                @pl.when(j != my)
                def _(j=j):
                    ag_recv(layer - 1, j).wait_send()

        for j in range(N_DEV):
            @pl.when(j != my)
            def _(j=j):
                r = rs_recv(2, j)
                r.wait_recv()
                r.wait_send()
        out_ref[...] = jnp.sum(rs_ref[2], axis=0)

    return pl.pallas_call(
        body,
        out_shape=jax.ShapeDtypeStruct((RB, D), jnp.float32),
        in_specs=[pl.BlockSpec(memory_space=pltpu.VMEM)] * 7,
        out_specs=pl.BlockSpec(memory_space=pltpu.VMEM),
        scratch_shapes=[
            pltpu.VMEM((B, D), jnp.float32),
            pltpu.VMEM((3, N_DEV, RB, D), jnp.float32),
            pltpu.VMEM((2, N_DEV, RB, D), jnp.float32),
            pltpu.SemaphoreType.DMA((3, N_DEV)),
            pltpu.SemaphoreType.DMA((3, N_DEV)),
            pltpu.SemaphoreType.DMA((2, N_DEV)),
            pltpu.SemaphoreType.DMA((2, N_DEV)),
        ],
        compiler_params=pltpu.CompilerParams(collective_id=0),
    )(x, Win0, Wout0, Win1, Wout1, Win2, Wout2)
